# Initial kernel scaffold; baseline (speedup 1.0000x reference)
import jax
import jax.numpy as jnp
from jax.experimental import pallas as pl


def kernel(tensor):
    raise NotImplementedError("write your pallas kernel here")


# speedup vs baseline: 27.2238x; 27.2238x over previous
import math

import jax
import jax.numpy as jnp
import numpy as np
from jax.experimental import pallas as pl

_SPARSITY = 0.05
_INT_MIN = np.int32(-(2**31))


def _kwta_body(x_ref, o_ref, *, k):
    x = x_ref[...]
    s = jax.lax.bitcast_convert_type(x, jnp.int32)
    ki = s ^ ((s >> 31) & np.int32(0x7FFFFFFF))
    rows = x.shape[0]
    p = jnp.zeros((rows, 1), jnp.int32)
    for bit in range(31, -1, -1):
        m = jnp.int32(np.uint32(1 << bit).astype(np.int32))
        pt = p | m
        thr = pt ^ _INT_MIN
        cnt = jnp.sum((ki >= thr).astype(jnp.int32), axis=1, keepdims=True)
        p = jnp.where(cnt >= k, pt, p)
    thr = p ^ _INT_MIN
    o_ref[...] = jnp.where(ki >= thr, x, 0.0)


def kernel(tensor):
    b, f, e = tensor.shape
    k = int(math.ceil(_SPARSITY * e))
    t = tensor.reshape(b * f, e)
    rows = b * f
    block_rows = 256
    grid = rows // block_rows
    out = pl.pallas_call(
        lambda x_ref, o_ref: _kwta_body(x_ref, o_ref, k=k),
        grid=(grid,),
        in_specs=[pl.BlockSpec((block_rows, e), lambda i: (i, 0))],
        out_specs=pl.BlockSpec((block_rows, e), lambda i: (i, 0)),
        out_shape=jax.ShapeDtypeStruct((rows, e), jnp.float32),
    )(t)
    return out.reshape(b, f, e)
